# baseline (speedup 1.0000x reference)
import functools

import jax
import jax.numpy as jnp
import numpy as np
from jax import lax
from jax.experimental import pallas as pl
from jax.experimental.pallas import tpu as pltpu
from jax.experimental.pallas import tpu_sc as plsc


def _repack_table(table):
    V, D = table.shape
    BLKV = 16384
    grid = (V + BLKV - 1) // BLKV
    P = BLKV // 4

    def body(tT_ref, eye_ref, out_ref):
        t = jax.lax.dot_general(
            tT_ref[...], eye_ref[...],
            dimension_numbers=(((0,), (0,)), ((), ())),
            preferred_element_type=jnp.float32,
        )
        tu = jax.lax.bitcast_convert_type(t, jnp.uint32)
        lo = tu[:, 0:D // 2] >> 16
        hi = tu[:, D // 2:D] & jnp.uint32(0xFFFF0000)
        tp = jax.lax.bitcast_convert_type(lo | hi, jnp.float32)
        for q in range(4):
            out_ref[:, q * (D // 2):(q + 1) * (D // 2)] = (
                tp[q * P:(q + 1) * P, :])

    packed = pl.pallas_call(
        body,
        grid=(grid,),
        in_specs=[
            pl.BlockSpec((D, BLKV), lambda i: (0, i)),
            pl.BlockSpec((D, D), lambda i: (0, 0)),
        ],
        out_specs=pl.BlockSpec((P, 2 * D), lambda i: (i, 0)),
        out_shape=jax.ShapeDtypeStruct((grid * P, 2 * D), jnp.float32),
        compiler_params=pltpu.CompilerParams(fuse_transposed_lhs_in_matmul=True),
    )(table.T, jnp.eye(D, dtype=jnp.float32))
    return packed.reshape(grid * BLKV, D // 2)


def _stage_index(x, BLKV=16384):
    P = BLKV // 4
    sh = P.bit_length() - 1
    return (x & ~(BLKV - 1)) + ((x & (P - 1)) << 2) + ((x >> sh) & 3)


def _embedding_bag_mean(x, table_r, D):
    B, H = x.shape
    W = table_r.shape[1]
    info = plsc.get_sparse_core_info()
    NC, NS, L = info.num_cores, info.num_subcores, info.num_lanes
    NW = NC * NS
    b_per_w = B // NW
    CHUNK = 2
    IPC = CHUNK * H
    n_chunks = b_per_w // CHUNK

    x_r = x.reshape(NW, n_chunks, IPC).astype(jnp.int32)
    mesh = plsc.VectorSubcoreMesh(core_axis_name="c", subcore_axis_name="s")

    @functools.partial(
        pl.kernel,
        mesh=mesh,
        out_type=jax.ShapeDtypeStruct((B, D), jnp.float32),
        scratch_types=[
            pltpu.VMEM((n_chunks, IPC), jnp.int32),
            pltpu.VMEM((8, IPC, W), jnp.float32),
            pltpu.VMEM((b_per_w, D), jnp.float32),
        ] + [pltpu.SemaphoreType.DMA] * 8,
        compiler_params=pltpu.CompilerParams(
            use_tc_tiling_on_sc=False, needs_layout_passes=False),
    )
    def emb_kernel(x_hbm, table_hbm, out_hbm, idx_v, rows_v, h_v, *sems):
        wid = lax.axis_index("s") * NC + lax.axis_index("c")
        pltpu.sync_copy(x_hbm.at[wid], idx_v)
        scale = jnp.float32(1.0 / H)

        def tree(vals):
            while len(vals) > 1:
                nxt = [vals[i] + vals[i + 1]
                       for i in range(0, len(vals) - 1, 2)]
                if len(vals) % 2:
                    nxt.append(vals[-1])
                vals = nxt
            return vals[0]

        def accum(c, slot):
            for j in range(CHUNK):
                for half in range(W // L):
                    vals = [
                        plsc.bitcast(
                            rows_v[slot, j * H + k, pl.ds(half * L, L)],
                            jnp.bfloat16)
                        for k in range(H)
                    ]
                    a, b = plsc.unpack(
                        tree(vals), format=plsc.PackFormat.INTERLEAVED)
                    h_v[c * CHUNK + j, pl.ds(2 * half * L, L)] = a * scale
                    h_v[c * CHUNK + j, pl.ds((2 * half + 1) * L, L)] = b * scale

        def fire(c, slot):
            pltpu.async_copy(table_hbm.at[idx_v.at[c]], rows_v.at[slot], sems[slot])

        def wait(c, slot):
            pltpu.make_async_copy(
                table_hbm.at[idx_v.at[c]], rows_v.at[slot], sems[slot]).wait()

        NBUF = 8
        for s in range(NBUF - 1):
            fire(s, s)

        def body(i, carry):
            c = NBUF * i
            for s in range(NBUF):
                cc = c + s
                wait(cc, s)
                accum(cc, s)
                nf = cc + NBUF - 1

                @pl.when(nf < n_chunks)
                def _():
                    fire(nf, (s + NBUF - 1) % NBUF)
            return carry

        lax.fori_loop(0, n_chunks // NBUF, body, 0)
        pltpu.sync_copy(h_v, out_hbm.at[pl.ds(wid * b_per_w, b_per_w)])

    return emb_kernel(x_r, table_r)


def _mlp_softmax(h, w1t, b1, w2t, b2):
    B, D = h.shape
    BLK = 2048

    def body(h_ref, w1_ref, b1_ref, w2_ref, b2_ref, o_ref):
        z = jnp.dot(h_ref[...], w1_ref[...], preferred_element_type=jnp.float32)
        z = z + b1_ref[...]
        z = jnp.dot(z, w2_ref[...], preferred_element_type=jnp.float32)
        z = z + b2_ref[...]
        z = z - jnp.max(z, axis=1, keepdims=True)
        e = jnp.exp(z)
        o_ref[...] = e / jnp.sum(e, axis=1, keepdims=True)

    return pl.pallas_call(
        body,
        grid=(B // BLK,),
        in_specs=[
            pl.BlockSpec((BLK, D), lambda i: (i, 0)),
            pl.BlockSpec((D, D), lambda i: (0, 0)),
            pl.BlockSpec((1, D), lambda i: (0, 0)),
            pl.BlockSpec((D, D), lambda i: (0, 0)),
            pl.BlockSpec((1, D), lambda i: (0, 0)),
        ],
        out_specs=pl.BlockSpec((BLK, D), lambda i: (i, 0)),
        out_shape=jax.ShapeDtypeStruct((B, D), jnp.float32),
    )(h, w1t, b1.reshape(1, D), w2t, b2.reshape(1, D))


_H_PERM = np.concatenate([
    np.arange(0, 16), np.arange(32, 48),
    np.arange(16, 32), np.arange(48, 64),
])


def kernel(x, table, W1, b1, W2, b2):
    table_r = _repack_table(table)
    h = _embedding_bag_mean(_stage_index(x), table_r, table.shape[1])
    return _mlp_softmax(h, W1.T[_H_PERM, :], b1, W2.T, b2)


# speedup vs baseline: 1.0426x; 1.0426x over previous
import functools

import jax
import jax.numpy as jnp
import numpy as np
from jax import lax
from jax.experimental import pallas as pl
from jax.experimental.pallas import tpu as pltpu
from jax.experimental.pallas import tpu_sc as plsc


def _repack_table(table):
    V, D = table.shape
    BLKV = 16384
    grid = (V + BLKV - 1) // BLKV
    P = BLKV // 4

    def body(tT_ref, eye_ref, out_ref):
        t = jax.lax.dot_general(
            tT_ref[...], eye_ref[...],
            dimension_numbers=(((0,), (0,)), ((), ())),
            preferred_element_type=jnp.float32,
        )
        tu = jax.lax.bitcast_convert_type(t, jnp.uint32)
        lo = tu[:, 0:D // 2] >> 16
        hi = tu[:, D // 2:D] & jnp.uint32(0xFFFF0000)
        tp = jax.lax.bitcast_convert_type(lo | hi, jnp.float32)
        for q in range(4):
            out_ref[:, q * (D // 2):(q + 1) * (D // 2)] = (
                tp[q * P:(q + 1) * P, :])

    packed = pl.pallas_call(
        body,
        grid=(grid,),
        in_specs=[
            pl.BlockSpec((D, BLKV), lambda i: (0, i)),
            pl.BlockSpec((D, D), lambda i: (0, 0)),
        ],
        out_specs=pl.BlockSpec((P, 2 * D), lambda i: (i, 0)),
        out_shape=jax.ShapeDtypeStruct((grid * P, 2 * D), jnp.float32),
        compiler_params=pltpu.CompilerParams(fuse_transposed_lhs_in_matmul=True),
    )(table.T, jnp.eye(D, dtype=jnp.float32))
    return packed.reshape(grid * BLKV, D // 2)


def _stage_index(x, BLKV=16384):
    P = BLKV // 4
    sh = P.bit_length() - 1
    return (x & ~(BLKV - 1)) + ((x & (P - 1)) << 2) + ((x >> sh) & 3)


def _embedding_bag_mean(x, table_r, D):
    B, H = x.shape
    W = table_r.shape[1]
    info = plsc.get_sparse_core_info()
    NC, NS, L = info.num_cores, info.num_subcores, info.num_lanes
    NW = NC * NS
    b_per_w = B // NW
    CHUNK = 2
    IPC = CHUNK * H
    n_chunks = b_per_w // CHUNK

    x_r = x.reshape(NW, n_chunks, IPC).astype(jnp.int32)
    mesh = plsc.VectorSubcoreMesh(core_axis_name="c", subcore_axis_name="s")

    @functools.partial(
        pl.kernel,
        mesh=mesh,
        out_type=jax.ShapeDtypeStruct((B, D), jnp.float32),
        scratch_types=[
            pltpu.VMEM((n_chunks, IPC), jnp.int32),
            pltpu.VMEM((4, IPC, W), jnp.float32),
            pltpu.VMEM((b_per_w, D), jnp.float32),
        ] + [pltpu.SemaphoreType.DMA] * 4,
        compiler_params=pltpu.CompilerParams(
            use_tc_tiling_on_sc=False, needs_layout_passes=False),
    )
    def emb_kernel(x_hbm, table_hbm, out_hbm, idx_v, rows_v, h_v, *sems):
        wid = lax.axis_index("s") * NC + lax.axis_index("c")
        pltpu.sync_copy(x_hbm.at[wid], idx_v)
        scale = jnp.float32(1.0 / H)

        def tree(vals):
            while len(vals) > 1:
                nxt = [vals[i] + vals[i + 1]
                       for i in range(0, len(vals) - 1, 2)]
                if len(vals) % 2:
                    nxt.append(vals[-1])
                vals = nxt
            return vals[0]

        def accum(c, slot):
            for j in range(CHUNK):
                for half in range(W // L):
                    vals = [
                        plsc.bitcast(
                            rows_v[slot, j * H + k, pl.ds(half * L, L)],
                            jnp.bfloat16)
                        for k in range(H)
                    ]
                    a, b = plsc.unpack(
                        tree(vals), format=plsc.PackFormat.INTERLEAVED)
                    h_v[c * CHUNK + j, pl.ds(2 * half * L, L)] = a * scale
                    h_v[c * CHUNK + j, pl.ds((2 * half + 1) * L, L)] = b * scale

        def fire(c, slot):
            pltpu.async_copy(table_hbm.at[idx_v.at[c]], rows_v.at[slot], sems[slot])

        def wait(c, slot):
            pltpu.make_async_copy(
                table_hbm.at[idx_v.at[c]], rows_v.at[slot], sems[slot]).wait()

        NBUF = 4
        for s in range(NBUF - 1):
            fire(s, s)

        def body(i, carry):
            c = NBUF * i
            for s in range(NBUF):
                cc = c + s
                wait(cc, s)
                accum(cc, s)
                nf = cc + NBUF - 1

                @pl.when(nf < n_chunks)
                def _():
                    fire(nf, (s + NBUF - 1) % NBUF)
            return carry

        lax.fori_loop(0, n_chunks // NBUF, body, 0)
        pltpu.sync_copy(h_v, out_hbm.at[pl.ds(wid * b_per_w, b_per_w)])

    return emb_kernel(x_r, table_r)


def _mlp_softmax(h, w1t, b1, w2t, b2):
    B, D = h.shape
    BLK = 2048

    def body(h_ref, w1_ref, b1_ref, w2_ref, b2_ref, o_ref):
        z = jnp.dot(h_ref[...], w1_ref[...], preferred_element_type=jnp.float32)
        z = z + b1_ref[...]
        z = jnp.dot(z, w2_ref[...], preferred_element_type=jnp.float32)
        z = z + b2_ref[...]
        z = z - jnp.max(z, axis=1, keepdims=True)
        e = jnp.exp(z)
        o_ref[...] = e / jnp.sum(e, axis=1, keepdims=True)

    return pl.pallas_call(
        body,
        grid=(B // BLK,),
        in_specs=[
            pl.BlockSpec((BLK, D), lambda i: (i, 0)),
            pl.BlockSpec((D, D), lambda i: (0, 0)),
            pl.BlockSpec((1, D), lambda i: (0, 0)),
            pl.BlockSpec((D, D), lambda i: (0, 0)),
            pl.BlockSpec((1, D), lambda i: (0, 0)),
        ],
        out_specs=pl.BlockSpec((BLK, D), lambda i: (i, 0)),
        out_shape=jax.ShapeDtypeStruct((B, D), jnp.float32),
    )(h, w1t, b1.reshape(1, D), w2t, b2.reshape(1, D))


_H_PERM = np.concatenate([
    np.arange(0, 16), np.arange(32, 48),
    np.arange(16, 32), np.arange(48, 64),
])


def kernel(x, table, W1, b1, W2, b2):
    table_r = _repack_table(table)
    h = _embedding_bag_mean(_stage_index(x), table_r, table.shape[1])
    return _mlp_softmax(h, W1.T[_H_PERM, :], b1, W2.T, b2)
